# baseline (speedup 1.0000x reference)
import functools

import jax
import jax.numpy as jnp
from jax import lax
from jax.experimental import pallas as pl
from jax.experimental.pallas import tpu as pltpu
from jax.experimental.pallas import tpu_sc as plsc

NUM_ROWS = 16384
SEQ = 50
DIM = 32
NC, NS = 2, 16
NW = NC * NS
R_PER_W = NUM_ROWS // NW
NBUF = 2
LANES = 16

_mesh = plsc.VectorSubcoreMesh(core_axis_name="c", subcore_axis_name="s")


@functools.partial(
    pl.kernel,
    mesh=_mesh,
    out_type=jax.ShapeDtypeStruct((SEQ, DIM, NUM_ROWS), jnp.float32),
    scratch_types=[
        pltpu.VMEM((SEQ, R_PER_W), jnp.int32),
        [pltpu.VMEM((R_PER_W, DIM), jnp.float32) for _ in range(NBUF)],
        [pltpu.VMEM((DIM, R_PER_W), jnp.float32) for _ in range(NBUF)],
        [pltpu.SemaphoreType.DMA for _ in range(NBUF)],
        [pltpu.SemaphoreType.DMA for _ in range(NBUF)],
    ],
    compiler_params=pltpu.CompilerParams(
        use_tc_tiling_on_sc=False, needs_layout_passes=False
    ),
)
def _gather_kernel(xt_hbm, table_hbm, out_hbm, idx_v, rows, planes, gsem, ssem):
    wid = lax.axis_index("s") * NC + lax.axis_index("c")
    r0 = wid * R_PER_W
    pltpu.sync_copy(xt_hbm.at[:, pl.ds(r0, R_PER_W)], idx_v)

    lane_iota = lax.iota(jnp.int32, LANES)
    _COL_IDS = [jnp.full((LANES,), d, jnp.int32) for d in range(DIM)]

    def gather_copy(c, b):
        return pltpu.make_async_copy(
            table_hbm.at[idx_v.at[c]], rows[b], gsem[b]
        )

    def store_copy(c, b):
        return pltpu.make_async_copy(
            planes[b], out_hbm.at[c, :, pl.ds(r0, R_PER_W)], ssem[b]
        )

    for b in range(NBUF):
        gather_copy(b, b).start()

    def col_body(cr, carry):
        for b in range(NBUF):
            c = cr * NBUF + b
            gather_copy(c, b).wait()

            @pl.when(c >= NBUF)
            def _():
                store_copy(c, b).wait()

            @plsc.parallel_loop(0, R_PER_W // LANES, unroll=4)
            def _(s):
                row_ids = s * LANES + lane_iota
                off = pl.multiple_of(s * LANES, LANES)
                for d0 in range(0, DIM, 8):
                    vs = [
                        plsc.load_gather(rows[b], [row_ids, _COL_IDS[d0 + k]])
                        for k in range(8)
                    ]
                    for k in range(8):
                        planes[b][d0 + k, pl.ds(off, LANES)] = vs[k]

            @pl.when(c + NBUF < SEQ)
            def _():
                gather_copy(c + NBUF, b).start()

            store_copy(c, b).start()

        return carry

    lax.fori_loop(0, SEQ // NBUF, col_body, 0)
    for b in range(NBUF):
        store_copy(SEQ - NBUF + b, b).wait()


def kernel(x, table):
    out = _gather_kernel(x.T, table)
    return out.transpose(2, 0, 1)


# speedup vs baseline: 1.0665x; 1.0665x over previous
import functools

import jax
import jax.numpy as jnp
from jax import lax
from jax.experimental import pallas as pl
from jax.experimental.pallas import tpu as pltpu
from jax.experimental.pallas import tpu_sc as plsc

NUM_ROWS = 16384
SEQ = 50
DIM = 32
NC, NS = 2, 16
NW = NC * NS
R_PER_W = NUM_ROWS // NW
NBUF = 2
LANES = 16

_mesh = plsc.VectorSubcoreMesh(core_axis_name="c", subcore_axis_name="s")


@functools.partial(
    pl.kernel,
    mesh=_mesh,
    out_type=jax.ShapeDtypeStruct((SEQ, DIM, NUM_ROWS), jnp.float32),
    scratch_types=[
        pltpu.VMEM((SEQ, R_PER_W), jnp.int32),
        [pltpu.VMEM((R_PER_W, DIM), jnp.float32) for _ in range(NBUF)],
        [pltpu.VMEM((DIM, R_PER_W), jnp.float32) for _ in range(NBUF)],
        [pltpu.SemaphoreType.DMA for _ in range(NBUF)],
        [pltpu.SemaphoreType.DMA for _ in range(NBUF)],
    ],
    compiler_params=pltpu.CompilerParams(
        use_tc_tiling_on_sc=False, needs_layout_passes=False
    ),
)
def _gather_kernel(xt_hbm, table_hbm, out_hbm, idx_v, rows, planes, gsem, ssem):
    wid = lax.axis_index("s") * NC + lax.axis_index("c")
    r0 = wid * R_PER_W
    pltpu.sync_copy(xt_hbm.at[:, pl.ds(r0, R_PER_W)], idx_v)

    lane_iota = lax.iota(jnp.int32, LANES)
    _COL_IDS = [jnp.full((LANES,), d, jnp.int32) for d in range(DIM)]

    def gather_copy(c, b):
        return pltpu.make_async_copy(
            table_hbm.at[idx_v.at[c]], rows[b], gsem[b]
        )

    def store_copy(c, b):
        return pltpu.make_async_copy(
            planes[b], out_hbm.at[c, :, pl.ds(r0, R_PER_W)], ssem[b]
        )

    for b in range(NBUF):
        gather_copy(b, b).start()

    def col_body(cr, carry):
        for b in range(NBUF):
            c = cr * NBUF + b
            gather_copy(c, b).wait()

            @pl.when(c >= NBUF)
            def _():
                store_copy(c, b).wait()

            @plsc.parallel_loop(0, R_PER_W // LANES, unroll=2)
            def _(s):
                row_ids = s * LANES + lane_iota
                off = pl.multiple_of(s * LANES, LANES)
                for d0 in range(0, DIM, 8):
                    vs = [
                        plsc.load_gather(rows[b], [row_ids, _COL_IDS[d0 + k]])
                        for k in range(8)
                    ]
                    for k in range(8):
                        planes[b][d0 + k, pl.ds(off, LANES)] = vs[k]

            @pl.when(c + NBUF < SEQ)
            def _():
                gather_copy(c + NBUF, b).start()

            store_copy(c, b).start()

        return carry

    lax.fori_loop(0, SEQ // NBUF, col_body, 0)
    for b in range(NBUF):
        store_copy(SEQ - NBUF + b, b).wait()


def kernel(x, table):
    out = _gather_kernel(x.T, table)
    return out.transpose(2, 0, 1)
